# Initial kernel scaffold; baseline (speedup 1.0000x reference)
import jax
import jax.numpy as jnp
from jax.experimental import pallas as pl


def kernel(x, rotation, eigenvalues, mean, centroids):
    raise NotImplementedError("write your pallas kernel here")


# speedup vs baseline: 5.4728x; 5.4728x over previous
import jax
import jax.numpy as jnp
from jax.experimental import pallas as pl
from jax.experimental.pallas import tpu as pltpu


def _body(x_ref, rott_ref, rot_ref, eig_ref, mean_ref, cent_ref,
          xhat_ref, idx_ref):
    d = rot_ref.shape[0]
    k = cent_ref.shape[0]
    eig = jnp.maximum(eig_ref[...], 1e-12)
    ws = jnp.sqrt((1.0 / d) / eig)
    inv_ws = 1.0 / ws

    xm = x_ref[...] - mean_ref[...]
    y = jnp.dot(xm, rott_ref[...],
                preferred_element_type=jnp.float32) * ws

    c0 = cent_ref[0]
    idx_f = jnp.zeros(y.shape, jnp.float32)
    y_hat = jnp.full(y.shape, c0, jnp.float32)
    for j in range(k - 1):
        cj, cj1 = cent_ref[j], cent_ref[j + 1]
        t = (y > 0.5 * (cj + cj1)).astype(jnp.float32)
        idx_f += t
        y_hat += t * (cj1 - cj)

    xhat_ref[...] = jnp.dot(y_hat * inv_ws, rot_ref[...],
                            preferred_element_type=jnp.float32) + mean_ref[...]
    idx_ref[...] = idx_f.astype(jnp.int32)


def kernel(x, rotation, eigenvalues, mean, centroids):
    n, d = x.shape
    tile = 1024
    grid = (n // tile,)
    rot_t = rotation.T
    eig2 = eigenvalues.reshape(1, d)
    mean2 = mean.reshape(1, d)

    x_hat, idx = pl.pallas_call(
        _body,
        grid=grid,
        in_specs=[
            pl.BlockSpec((tile, d), lambda i: (i, 0)),
            pl.BlockSpec((d, d), lambda i: (0, 0)),
            pl.BlockSpec((d, d), lambda i: (0, 0)),
            pl.BlockSpec((1, d), lambda i: (0, 0)),
            pl.BlockSpec((1, d), lambda i: (0, 0)),
            pl.BlockSpec(memory_space=pltpu.SMEM),
        ],
        out_specs=[
            pl.BlockSpec((tile, d), lambda i: (i, 0)),
            pl.BlockSpec((tile, d), lambda i: (i, 0)),
        ],
        out_shape=[
            jax.ShapeDtypeStruct((n, d), jnp.float32),
            jax.ShapeDtypeStruct((n, d), jnp.int32),
        ],
    )(x, rot_t, rotation, eig2, mean2, centroids)
    return x_hat, idx


# speedup vs baseline: 7.1869x; 1.3132x over previous
import jax
import jax.numpy as jnp
from jax.experimental import pallas as pl
from jax.experimental.pallas import tpu as pltpu


def _body(x_ref, rott_ref, rot_ref, eig_ref, mean_ref, cent_ref,
          xhat_ref, idx_ref):
    d = rot_ref.shape[0]
    k = cent_ref.shape[0]
    eig = jnp.maximum(eig_ref[...], 1e-12)
    ws = jnp.sqrt((1.0 / d) / eig)
    inv_ws = 1.0 / ws

    yr = jnp.dot(x_ref[...] - mean_ref[...], rott_ref[...],
                 preferred_element_type=jnp.float32)

    w = cent_ref[0] * inv_ws
    for j in range(k - 1):
        bvec = (0.5 * (cent_ref[j] + cent_ref[j + 1])) * inv_ws
        kvec = 16.0 * (j + 1) + cent_ref[j + 1] * inv_ws
        w = jnp.where(yr > bvec, kvec, w)

    qf = jnp.floor(w * (1.0 / 16.0) + 0.5)
    yh_scaled = w - 16.0 * qf

    xhat_ref[...] = jnp.dot(yh_scaled, rot_ref[...],
                            preferred_element_type=jnp.float32) + mean_ref[...]
    idx_ref[...] = qf.astype(jnp.int32)


def kernel(x, rotation, eigenvalues, mean, centroids):
    n, d = x.shape
    tile = 1024
    grid = (n // tile,)
    rot_t = rotation.T
    eig2 = eigenvalues.reshape(1, d)
    mean2 = mean.reshape(1, d)

    x_hat, idx = pl.pallas_call(
        _body,
        grid=grid,
        in_specs=[
            pl.BlockSpec((tile, d), lambda i: (i, 0)),
            pl.BlockSpec((d, d), lambda i: (0, 0)),
            pl.BlockSpec((d, d), lambda i: (0, 0)),
            pl.BlockSpec((1, d), lambda i: (0, 0)),
            pl.BlockSpec((1, d), lambda i: (0, 0)),
            pl.BlockSpec(memory_space=pltpu.SMEM),
        ],
        out_specs=[
            pl.BlockSpec((tile, d), lambda i: (i, 0)),
            pl.BlockSpec((tile, d), lambda i: (i, 0)),
        ],
        out_shape=[
            jax.ShapeDtypeStruct((n, d), jnp.float32),
            jax.ShapeDtypeStruct((n, d), jnp.int32),
        ],
    )(x, rot_t, rotation, eig2, mean2, centroids)
    return x_hat, idx


# speedup vs baseline: 8.7593x; 1.2188x over previous
import jax
import jax.numpy as jnp
from jax.experimental import pallas as pl
from jax.experimental.pallas import tpu as pltpu


def _body(x_ref, rott_ref, rot_ref, eig_ref, mean_ref, cent_ref,
          xhat_ref, idx_ref):
    d = rot_ref.shape[0]
    k = cent_ref.shape[0]
    eig = jnp.maximum(eig_ref[...], 1e-12)
    ws = jnp.sqrt((1.0 / d) / eig)
    inv_ws = 1.0 / ws

    yr = jnp.dot(x_ref[...] - mean_ref[...], rott_ref[...],
                 preferred_element_type=jnp.float32)

    def bvec(j):
        return (0.5 * (cent_ref[j] + cent_ref[j + 1])) * inv_ws

    def kvec(j):
        return 16.0 * j + cent_ref[j] * inv_ws

    half = k // 2
    w_lo = kvec(0)
    w_hi = kvec(half)
    for j in range(half - 1):
        w_lo = jnp.where(yr > bvec(j), kvec(j + 1), w_lo)
        w_hi = jnp.where(yr > bvec(half + j), kvec(half + j + 1), w_hi)
    w = jnp.where(yr > bvec(half - 1), w_hi, w_lo)

    qf = jnp.floor(w * (1.0 / 16.0) + 0.5)
    yh_scaled = w - 16.0 * qf

    xhat_ref[...] = jnp.dot(yh_scaled, rot_ref[...],
                            preferred_element_type=jnp.float32) + mean_ref[...]
    idx_ref[...] = qf.astype(jnp.int32)


def kernel(x, rotation, eigenvalues, mean, centroids):
    n, d = x.shape
    tile = 2048
    grid = (n // tile,)
    rot_t = rotation.T
    eig2 = eigenvalues.reshape(1, d)
    mean2 = mean.reshape(1, d)

    x_hat, idx = pl.pallas_call(
        _body,
        grid=grid,
        in_specs=[
            pl.BlockSpec((tile, d), lambda i: (i, 0)),
            pl.BlockSpec((d, d), lambda i: (0, 0)),
            pl.BlockSpec((d, d), lambda i: (0, 0)),
            pl.BlockSpec((1, d), lambda i: (0, 0)),
            pl.BlockSpec((1, d), lambda i: (0, 0)),
            pl.BlockSpec(memory_space=pltpu.SMEM),
        ],
        out_specs=[
            pl.BlockSpec((tile, d), lambda i: (i, 0)),
            pl.BlockSpec((tile, d), lambda i: (i, 0)),
        ],
        out_shape=[
            jax.ShapeDtypeStruct((n, d), jnp.float32),
            jax.ShapeDtypeStruct((n, d), jnp.int32),
        ],
    )(x, rot_t, rotation, eig2, mean2, centroids)
    return x_hat, idx


# speedup vs baseline: 8.9031x; 1.0164x over previous
import jax
import jax.numpy as jnp
from jax.experimental import pallas as pl
from jax.experimental.pallas import tpu as pltpu

_MAGIC = 8388608.0
_MAGIC_BITS = 0x4B000000


def _body(x_ref, rott_ref, rot_ref, eig_ref, mean_ref, cent_ref,
          xhat_ref, idx_ref, b_scr, k_scr):
    d = rot_ref.shape[0]
    k = 16

    @pl.when(pl.program_id(0) == 0)
    def _init():
        eig = jnp.maximum(eig_ref[...], 1e-12)
        inv_ws = 1.0 / jnp.sqrt((1.0 / d) / eig)
        mrow = jnp.dot(mean_ref[...], rott_ref[...],
                       preferred_element_type=jnp.float32)
        for j in range(k):
            k_scr[j:j + 1, :] = 16.0 * j + cent_ref[j] * inv_ws + mrow
            jn = min(j + 1, k - 1)
            b_scr[j:j + 1, :] = (0.5 * (cent_ref[j] + cent_ref[jn])) * inv_ws + mrow

    yr = jnp.dot(x_ref[...], rott_ref[...],
                 preferred_element_type=jnp.float32)

    half = k // 2
    w_lo = k_scr[0:1, :]
    w_hi = k_scr[half:half + 1, :]
    for j in range(half - 1):
        w_lo = jnp.where(yr > b_scr[j:j + 1, :], k_scr[j + 1:j + 2, :], w_lo)
        jh = half + j
        w_hi = jnp.where(yr > b_scr[jh:jh + 1, :], k_scr[jh + 1:jh + 2, :], w_hi)
    w = jnp.where(yr > b_scr[half - 1:half, :], w_hi, w_lo)

    qm = w * (1.0 / 16.0) + _MAGIC
    idx_i32 = (jax.lax.bitcast_convert_type(qm, jnp.int32)
               - jnp.int32(_MAGIC_BITS))
    qf = idx_i32.astype(jnp.float32)
    yh = w - 16.0 * qf

    xhat_ref[...] = jnp.dot(yh, rot_ref[...],
                            preferred_element_type=jnp.float32)
    idx_ref[...] = idx_i32


def kernel(x, rotation, eigenvalues, mean, centroids):
    n, d = x.shape
    tile = 2048
    grid = (n // tile,)
    rot_t = rotation.T
    eig2 = eigenvalues.reshape(1, d)
    mean2 = mean.reshape(1, d)

    x_hat, idx = pl.pallas_call(
        _body,
        grid=grid,
        in_specs=[
            pl.BlockSpec((tile, d), lambda i: (i, 0)),
            pl.BlockSpec((d, d), lambda i: (0, 0)),
            pl.BlockSpec((d, d), lambda i: (0, 0)),
            pl.BlockSpec((1, d), lambda i: (0, 0)),
            pl.BlockSpec((1, d), lambda i: (0, 0)),
            pl.BlockSpec(memory_space=pltpu.SMEM),
        ],
        out_specs=[
            pl.BlockSpec((tile, d), lambda i: (i, 0)),
            pl.BlockSpec((tile, d), lambda i: (i, 0)),
        ],
        out_shape=[
            jax.ShapeDtypeStruct((n, d), jnp.float32),
            jax.ShapeDtypeStruct((n, d), jnp.int32),
        ],
        scratch_shapes=[
            pltpu.VMEM((16, 128), jnp.float32),
            pltpu.VMEM((16, 128), jnp.float32),
        ],
    )(x, rot_t, rotation, eig2, mean2, centroids)
    return x_hat, idx
